# baseline (speedup 1.0000x reference)
import functools

import jax
import jax.numpy as jnp
from jax import lax
from jax.experimental import pallas as pl
from jax.experimental.pallas import tpu as pltpu
from jax.experimental.pallas import tpu_sc as plsc

NC = 2
NS = 16
NW = NC * NS
CH = 128


def _mesh():
    return plsc.VectorSubcoreMesh(core_axis_name="c", subcore_axis_name="s")


def _zero_stripe(zero_hbm, dst_ref, base, stripe):
    full, rem = divmod(stripe, 128)
    for t in range(full):
        pltpu.sync_copy(zero_hbm, dst_ref.at[pl.ds(base + t * 128, 128)])
    if rem:
        pltpu.sync_copy(zero_hbm.at[pl.ds(0, rem)],
                        dst_ref.at[pl.ds(base + full * 128, rem)])


def _sc_hist(dst2d, zeros_n, n_pad):
    n_chunks = dst2d.shape[0]
    cpw = n_chunks // NW

    @functools.partial(
        pl.kernel,
        out_type=jax.ShapeDtypeStruct((NW * n_pad,), jnp.float32),
        mesh=_mesh(),
        compiler_params=pltpu.CompilerParams(needs_layout_passes=False),
        scratch_types=[
            pltpu.VMEM((n_pad,), jnp.float32),
            pltpu.VMEM((cpw, CH), jnp.int32),
        ],
    )
    def k(dst_hbm, z_hbm, out_hbm, hist, idx_v):
        c = lax.axis_index("c")
        s = lax.axis_index("s")
        wid = s * NC + c
        pltpu.sync_copy(z_hbm, hist)
        pltpu.sync_copy(dst_hbm.at[pl.ds(wid * cpw, cpw)], idx_v)
        ones = jnp.ones((16,), jnp.float32)

        def body(r, carry):
            for k16 in range(0, CH, 16):
                iv = idx_v[r, pl.ds(k16, 16)]
                plsc.addupdate_scatter(hist, [iv], ones)
            return carry

        lax.fori_loop(0, cpw, body, 0)
        pltpu.sync_copy(hist, out_hbm.at[pl.ds(wid * n_pad, n_pad)])

    return k(dst2d, zeros_n)


def _sc_scatter(h2, src2d, dst2d, zeros128, n_pad):
    n, d = h2.shape
    n_chunks = src2d.shape[0]
    cpw = n_chunks // NW
    stripe = n_pad // NS

    hpw = cpw // 2

    @functools.partial(
        pl.kernel,
        out_type=jax.ShapeDtypeStruct((NC * NS, stripe, d), jnp.float32),
        mesh=_mesh(),
        scratch_types=[
            pltpu.VMEM_SHARED((n_pad, d), jnp.float32),
            pltpu.VMEM((hpw, CH), jnp.int32),
            pltpu.VMEM((hpw, CH), jnp.int32),
            pltpu.VMEM((CH, d), jnp.float32),
            pltpu.VMEM((CH, d), jnp.float32),
            pltpu.SemaphoreType.DMA,
            pltpu.SemaphoreType.DMA,
            pltpu.SemaphoreType.DMA,
            pltpu.SemaphoreType.DMA,
        ],
    )
    def k(h2_hbm, src_hbm, dst_hbm, z_hbm, out_hbm,
          agg, isrc, idst, buf_a, buf_b, sem_a, sem_b, ssem_a, ssem_b):
        c = lax.axis_index("c")
        s = lax.axis_index("s")
        wid = s * NC + c
        base = s * stripe
        _zero_stripe(z_hbm, agg, base, stripe)
        plsc.subcore_barrier()

        for ph in range(2):
            off = wid * cpw + ph * hpw
            pltpu.sync_copy(src_hbm.at[pl.ds(off, hpw)], isrc)
            pltpu.sync_copy(dst_hbm.at[pl.ds(off, hpw)], idst)
            pltpu.async_copy(h2_hbm.at[isrc.at[0]], buf_a, sem_a)
            pltpu.async_copy(h2_hbm.at[isrc.at[1]], buf_b, sem_b)

            def body(j2, carry):
                ja = 2 * j2
                jb = ja + 1
                pltpu.make_async_copy(h2_hbm.at[isrc.at[ja]], buf_a,
                                      sem_a).wait()
                pltpu.async_copy(buf_a, agg.at[idst.at[ja]], ssem_a, add=True)

                pltpu.make_async_copy(h2_hbm.at[isrc.at[jb]], buf_b,
                                      sem_b).wait()
                pltpu.async_copy(buf_b, agg.at[idst.at[jb]], ssem_b, add=True)

                @pl.when(ja + 2 < hpw)
                def _():
                    pltpu.make_async_copy(buf_a, agg.at[idst.at[ja]],
                                          ssem_a).wait()
                    pltpu.async_copy(h2_hbm.at[isrc.at[ja + 2]], buf_a, sem_a)

                @pl.when(jb + 2 < hpw)
                def _():
                    pltpu.make_async_copy(buf_b, agg.at[idst.at[jb]],
                                          ssem_b).wait()
                    pltpu.async_copy(h2_hbm.at[isrc.at[jb + 2]], buf_b, sem_b)

                return carry

            lax.fori_loop(0, hpw // 2, body, 0)
            pltpu.make_async_copy(buf_a, agg.at[idst.at[hpw - 2]],
                                  ssem_a).wait()
            pltpu.make_async_copy(buf_b, agg.at[idst.at[hpw - 1]],
                                  ssem_b).wait()
        plsc.subcore_barrier()
        pltpu.sync_copy(agg.at[pl.ds(base, stripe)], out_hbm.at[c * NS + s])

    out = k(h2, src2d, dst2d, zeros128)
    return out.reshape(NC, n_pad, d)


def _tc_h2(x, w, hist, blk):
    n, d = x.shape

    def body(x_ref, w_ref, hist_ref, h2_ref, dinv_ref):
        deg = lax.dot_general(hist_ref[...], jnp.ones((NW, 1), jnp.float32),
                              (((0,), (0,)), ((), ())),
                              preferred_element_type=jnp.float32) + 1.0
        dinv = lax.rsqrt(deg)
        dinv_ref[...] = jnp.broadcast_to(dinv, dinv_ref.shape)
        h = jnp.dot(x_ref[...], w_ref[...], preferred_element_type=jnp.float32)
        h2_ref[...] = h * dinv

    return pl.pallas_call(
        body,
        grid=((n + blk - 1) // blk,),
        in_specs=[
            pl.BlockSpec((blk, d), lambda i: (i, 0)),
            pl.BlockSpec((d, d), lambda i: (0, 0)),
            pl.BlockSpec((NW, blk), lambda i: (0, i)),
        ],
        out_specs=[
            pl.BlockSpec((blk, d), lambda i: (i, 0)),
            pl.BlockSpec((blk, 16), lambda i: (i, 0)),
        ],
        out_shape=[
            jax.ShapeDtypeStruct((n, d), jnp.float32),
            jax.ShapeDtypeStruct((n, 16), jnp.float32),
        ],
    )(x, w, hist)


def _tc_pre(parts, h2, dinv, b2, blk):
    n, d = h2.shape

    def body(p_ref, h2_ref, dinv_ref, b_ref, op_ref, st_ref):
        blk_rows = h2_ref.shape[0]
        tot = p_ref[0] + p_ref[1] + h2_ref[...]
        op = tot * dinv_ref[:, 0:1] + b_ref[...]
        op_ref[...] = op

        @pl.when(pl.program_id(0) == 0)
        def _():
            st_ref[...] = jnp.zeros_like(st_ref)

        rid = (lax.broadcasted_iota(jnp.int32, (blk_rows, 1), 0)
               + pl.program_id(0) * blk_rows)
        opm = jnp.where(rid < n, op, 0.0)
        st_ref[0:1, :] += jnp.sum(opm, axis=0, keepdims=True)
        st_ref[1:2, :] += jnp.sum(opm * opm, axis=0, keepdims=True)

    return pl.pallas_call(
        body,
        grid=((n + blk - 1) // blk,),
        in_specs=[
            pl.BlockSpec((2, blk, d), lambda i: (0, i, 0)),
            pl.BlockSpec((blk, d), lambda i: (i, 0)),
            pl.BlockSpec((blk, 16), lambda i: (i, 0)),
            pl.BlockSpec((1, d), lambda i: (0, 0)),
        ],
        out_specs=[
            pl.BlockSpec((blk, d), lambda i: (i, 0)),
            pl.BlockSpec((8, d), lambda i: (0, 0)),
        ],
        out_shape=[
            jax.ShapeDtypeStruct((n, d), jnp.float32),
            jax.ShapeDtypeStruct((8, d), jnp.float32),
        ],
    )(parts, h2, dinv, b2)


def _tc_fin(out_pre, stats, x, bnw2, bnb2, blk):
    n, d = x.shape
    inv_n = 1.0 / n

    def body(op_ref, st_ref, x_ref, w_ref, b_ref, out_ref):
        mean = st_ref[0:1, :] * inv_n
        var = st_ref[1:2, :] * inv_n - mean * mean
        rstd = lax.rsqrt(var + 1e-5)
        y = (op_ref[...] - mean) * (rstd * w_ref[...]) + b_ref[...]
        out_ref[...] = jnp.maximum(y, 0.0) + x_ref[...]

    return pl.pallas_call(
        body,
        grid=((n + blk - 1) // blk,),
        in_specs=[
            pl.BlockSpec((blk, d), lambda i: (i, 0)),
            pl.BlockSpec((8, d), lambda i: (0, 0)),
            pl.BlockSpec((blk, d), lambda i: (i, 0)),
            pl.BlockSpec((1, d), lambda i: (0, 0)),
            pl.BlockSpec((1, d), lambda i: (0, 0)),
        ],
        out_specs=pl.BlockSpec((blk, d), lambda i: (i, 0)),
        out_shape=jax.ShapeDtypeStruct((n, d), jnp.float32),
    )(out_pre, stats, x, bnw2, bnb2)


def kernel(x, edge_index, W, b, bn_weight, bn_bias):
    n, d = x.shape
    e = edge_index.shape[1]

    cpw = 8 * ((e + NW * CH * 8 - 1) // (NW * CH * 8))
    e_pad = cpw * NW * CH
    pad = e_pad - e
    n_pad = 128 * ((n + 127) // 128) + 128

    src = edge_index[0].astype(jnp.int32)
    dst = edge_index[1].astype(jnp.int32)
    pad_src = jnp.arange(pad, dtype=jnp.int32) % jnp.int32(n)
    src2d = jnp.concatenate([src, pad_src]).reshape(e_pad // CH, CH)
    spare = n_pad - n
    pad_dst = n + jnp.arange(pad, dtype=jnp.int32) % spare
    dst2d = jnp.concatenate([dst, pad_dst]).reshape(e_pad // CH, CH)
    zeros128 = jnp.zeros((128, d), jnp.float32)
    zeros_n = jnp.zeros((n_pad,), jnp.float32)

    blk = 1024

    hist = _sc_hist(dst2d, zeros_n, n_pad).reshape(NW, n_pad)
    h2, dinv = _tc_h2(x, W, hist, blk)
    parts = _sc_scatter(h2, src2d, dst2d, zeros128, n_pad)
    out_pre, stats = _tc_pre(parts, h2, dinv, b.reshape(1, d), blk)
    return _tc_fin(out_pre, stats, x, bn_weight.reshape(1, d),
                   bn_bias.reshape(1, d), blk)


# speedup vs baseline: 1.2634x; 1.2634x over previous
import functools

import jax
import jax.numpy as jnp
from jax import lax
from jax.experimental import pallas as pl
from jax.experimental.pallas import tpu as pltpu
from jax.experimental.pallas import tpu_sc as plsc

NC = 2
NS = 16
NW = NC * NS
CH = 128


def _mesh():
    return plsc.VectorSubcoreMesh(core_axis_name="c", subcore_axis_name="s")


def _zero_stripe(zero_hbm, dst_ref, base, stripe):
    full, rem = divmod(stripe, 128)
    for t in range(full):
        pltpu.sync_copy(zero_hbm, dst_ref.at[pl.ds(base + t * 128, 128)])
    if rem:
        pltpu.sync_copy(zero_hbm.at[pl.ds(0, rem)],
                        dst_ref.at[pl.ds(base + full * 128, rem)])


def _sc_hist(edges3, zeros_n, n_pad):
    n_chunks = edges3.shape[1]
    cpw = n_chunks // NW

    @functools.partial(
        pl.kernel,
        out_type=jax.ShapeDtypeStruct((NW * n_pad,), jnp.float32),
        mesh=_mesh(),
        compiler_params=pltpu.CompilerParams(needs_layout_passes=False),
        scratch_types=[
            pltpu.VMEM((n_pad,), jnp.float32),
            pltpu.VMEM((cpw, CH), jnp.int32),
        ],
    )
    def k(edges_hbm, z_hbm, out_hbm, hist, idx_v):
        c = lax.axis_index("c")
        s = lax.axis_index("s")
        wid = s * NC + c
        pltpu.sync_copy(z_hbm, hist)
        pltpu.sync_copy(edges_hbm.at[1, pl.ds(wid * cpw, cpw)], idx_v)
        ones = jnp.ones((16,), jnp.float32)

        def body(r, carry):
            for k16 in range(0, CH, 16):
                iv = idx_v[r, pl.ds(k16, 16)]
                plsc.addupdate_scatter(hist, [iv], ones)
            return carry

        lax.fori_loop(0, cpw, body, 0)
        pltpu.sync_copy(hist, out_hbm.at[pl.ds(wid * n_pad, n_pad)])

    return k(edges3, zeros_n)


def _sc_scatter(h2, edges3, zeros128, n_pad):
    n, d = h2.shape
    n_chunks = edges3.shape[1]
    cpw = n_chunks // NW
    stripe = n_pad // NS

    hpw = cpw // 2

    @functools.partial(
        pl.kernel,
        out_type=jax.ShapeDtypeStruct((NC * NS, stripe, d), jnp.float32),
        mesh=_mesh(),
        scratch_types=[
            pltpu.VMEM_SHARED((n_pad, d), jnp.float32),
            pltpu.VMEM((hpw, CH), jnp.int32),
            pltpu.VMEM((hpw, CH), jnp.int32),
            pltpu.VMEM((CH, d), jnp.float32),
            pltpu.VMEM((CH, d), jnp.float32),
            pltpu.SemaphoreType.DMA,
            pltpu.SemaphoreType.DMA,
        ],
    )
    def k(h2_hbm, edges_hbm, z_hbm, out_hbm,
          agg, isrc, idst, buf_a, buf_b, sem_a, sem_b):
        c = lax.axis_index("c")
        s = lax.axis_index("s")
        wid = s * NC + c
        base = s * stripe
        _zero_stripe(z_hbm, agg, base, stripe)
        plsc.subcore_barrier()

        for ph in range(2):
            off = wid * cpw + ph * hpw
            pltpu.sync_copy(edges_hbm.at[0, pl.ds(off, hpw)], isrc)
            pltpu.sync_copy(edges_hbm.at[1, pl.ds(off, hpw)], idst)
            pltpu.async_copy(h2_hbm.at[isrc.at[0]], buf_a, sem_a)
            pltpu.async_copy(h2_hbm.at[isrc.at[1]], buf_b, sem_b)

            def body(j2, carry):
                ja = 2 * j2
                jb = ja + 1
                pltpu.make_async_copy(h2_hbm.at[isrc.at[ja]], buf_a,
                                      sem_a).wait()
                pltpu.sync_copy(buf_a, agg.at[idst.at[ja]], add=True)

                @pl.when(ja + 2 < hpw)
                def _():
                    pltpu.async_copy(h2_hbm.at[isrc.at[ja + 2]], buf_a, sem_a)

                pltpu.make_async_copy(h2_hbm.at[isrc.at[jb]], buf_b,
                                      sem_b).wait()
                pltpu.sync_copy(buf_b, agg.at[idst.at[jb]], add=True)

                @pl.when(jb + 2 < hpw)
                def _():
                    pltpu.async_copy(h2_hbm.at[isrc.at[jb + 2]], buf_b, sem_b)

                return carry

            lax.fori_loop(0, hpw // 2, body, 0)
        plsc.subcore_barrier()
        pltpu.sync_copy(agg.at[pl.ds(base, stripe)], out_hbm.at[c * NS + s])

    out = k(h2, edges3, zeros128)
    return out.reshape(NC, n_pad, d)


def _tc_h2(x, w, hist, blk):
    n, d = x.shape

    def body(x_ref, w_ref, hist_ref, h2_ref, dinv_ref):
        deg = lax.dot_general(hist_ref[...], jnp.ones((NW, 1), jnp.float32),
                              (((0,), (0,)), ((), ())),
                              preferred_element_type=jnp.float32) + 1.0
        dinv = lax.rsqrt(deg)
        dinv_ref[...] = jnp.broadcast_to(dinv, dinv_ref.shape)
        h = jnp.dot(x_ref[...], w_ref[...], preferred_element_type=jnp.float32)
        h2_ref[...] = h * dinv

    return pl.pallas_call(
        body,
        grid=((n + blk - 1) // blk,),
        in_specs=[
            pl.BlockSpec((blk, d), lambda i: (i, 0)),
            pl.BlockSpec((d, d), lambda i: (0, 0)),
            pl.BlockSpec((NW, blk), lambda i: (0, i)),
        ],
        out_specs=[
            pl.BlockSpec((blk, d), lambda i: (i, 0)),
            pl.BlockSpec((blk, 16), lambda i: (i, 0)),
        ],
        out_shape=[
            jax.ShapeDtypeStruct((n, d), jnp.float32),
            jax.ShapeDtypeStruct((n, 16), jnp.float32),
        ],
    )(x, w, hist)


def _tc_pre(parts, h2, dinv, b2, blk):
    n, d = h2.shape

    def body(p_ref, h2_ref, dinv_ref, b_ref, op_ref, st_ref):
        blk_rows = h2_ref.shape[0]
        tot = p_ref[0] + p_ref[1] + h2_ref[...]
        op = tot * dinv_ref[:, 0:1] + b_ref[...]
        op_ref[...] = op

        @pl.when(pl.program_id(0) == 0)
        def _():
            st_ref[...] = jnp.zeros_like(st_ref)

        rid = (lax.broadcasted_iota(jnp.int32, (blk_rows, 1), 0)
               + pl.program_id(0) * blk_rows)
        opm = jnp.where(rid < n, op, 0.0)
        st_ref[0:1, :] += jnp.sum(opm, axis=0, keepdims=True)
        st_ref[1:2, :] += jnp.sum(opm * opm, axis=0, keepdims=True)

    return pl.pallas_call(
        body,
        grid=((n + blk - 1) // blk,),
        in_specs=[
            pl.BlockSpec((2, blk, d), lambda i: (0, i, 0)),
            pl.BlockSpec((blk, d), lambda i: (i, 0)),
            pl.BlockSpec((blk, 16), lambda i: (i, 0)),
            pl.BlockSpec((1, d), lambda i: (0, 0)),
        ],
        out_specs=[
            pl.BlockSpec((blk, d), lambda i: (i, 0)),
            pl.BlockSpec((8, d), lambda i: (0, 0)),
        ],
        out_shape=[
            jax.ShapeDtypeStruct((n, d), jnp.float32),
            jax.ShapeDtypeStruct((8, d), jnp.float32),
        ],
    )(parts, h2, dinv, b2)


def _tc_fin(out_pre, stats, x, bnw2, bnb2, blk):
    n, d = x.shape
    inv_n = 1.0 / n

    def body(op_ref, st_ref, x_ref, w_ref, b_ref, out_ref):
        mean = st_ref[0:1, :] * inv_n
        var = st_ref[1:2, :] * inv_n - mean * mean
        rstd = lax.rsqrt(var + 1e-5)
        y = (op_ref[...] - mean) * (rstd * w_ref[...]) + b_ref[...]
        out_ref[...] = jnp.maximum(y, 0.0) + x_ref[...]

    return pl.pallas_call(
        body,
        grid=((n + blk - 1) // blk,),
        in_specs=[
            pl.BlockSpec((blk, d), lambda i: (i, 0)),
            pl.BlockSpec((8, d), lambda i: (0, 0)),
            pl.BlockSpec((blk, d), lambda i: (i, 0)),
            pl.BlockSpec((1, d), lambda i: (0, 0)),
            pl.BlockSpec((1, d), lambda i: (0, 0)),
        ],
        out_specs=pl.BlockSpec((blk, d), lambda i: (i, 0)),
        out_shape=jax.ShapeDtypeStruct((n, d), jnp.float32),
    )(out_pre, stats, x, bnw2, bnb2)


def kernel(x, edge_index, W, b, bn_weight, bn_bias):
    n, d = x.shape
    e = edge_index.shape[1]

    cpw = 8 * ((e + NW * CH * 8 - 1) // (NW * CH * 8))
    e_pad = cpw * NW * CH
    pad = e_pad - e
    n_pad = 128 * ((n + 127) // 128) + 128

    pad_src = jnp.arange(pad, dtype=jnp.int32) % jnp.int32(n)
    pad_dst = n + jnp.arange(pad, dtype=jnp.int32) % (n_pad - n)
    edges3 = jnp.concatenate(
        [edge_index.astype(jnp.int32), jnp.stack([pad_src, pad_dst])],
        axis=1).reshape(2, e_pad // CH, CH)
    zeros128 = jnp.zeros((128, d), jnp.float32)
    zeros_n = jnp.zeros((n_pad,), jnp.float32)

    blk = 1024

    hist = _sc_hist(edges3, zeros_n, n_pad).reshape(NW, n_pad)
    h2, dinv = _tc_h2(x, W, hist, blk)
    parts = _sc_scatter(h2, edges3, zeros128, n_pad)
    out_pre, stats = _tc_pre(parts, h2, dinv, b.reshape(1, d), blk)
    return _tc_fin(out_pre, stats, x, bn_weight.reshape(1, d),
                   bn_bias.reshape(1, d), blk)
